# baseline (speedup 1.0000x reference)
import functools

import jax
import jax.numpy as jnp
from jax import lax
from jax.experimental import pallas as pl
from jax.experimental.pallas import tpu as pltpu
from jax.experimental.pallas import tpu_sc as plsc

B = 4
C = 384
HF = 37
WF = 37
R = 512
P = 49
IMG_W = 518
PATCH = 7

NC = 2
NS = 16
NW = NC * NS
L = 16

SAMP_TOTAL = B * R
SAMP_PER_W = SAMP_TOTAL // NW
ROWS_PER_W = SAMP_PER_W * P
NBUF = 4
ROWS_PER_CHUNK = P
NCHUNK = SAMP_PER_W
W_PER_B = R // SAMP_PER_W

MAGIC = 12582912.0


def _sc_body(table_hbm, sidx_hbm, out_hbm, sidx_v, idx_buf, buf0, buf1,
             buf2, buf3, gsem, osem):
    wid = lax.axis_index("s") * NC + lax.axis_index("c")
    tbase = (wid // W_PER_B) * (HF * WF)
    s0 = wid * SAMP_PER_W
    out0 = wid * ROWS_PER_W

    pltpu.sync_copy(sidx_hbm.at[pl.ds(s0, SAMP_PER_W)], sidx_v)

    lanes = lax.iota(jnp.int32, L)

    def rnd(y):
        return ((y + jnp.float32(MAGIC)) - jnp.float32(MAGIC)).astype(
            jnp.int32)

    for g in range(SAMP_PER_W // L):
        s16 = sidx_v[pl.ds(g * L, L)]
        sf = s16.astype(jnp.float32)
        ypf = (sf / jnp.float32(IMG_W)).astype(jnp.int32).astype(jnp.float32)
        xpf = sf - ypf * jnp.float32(IMG_W)
        yfeat = jnp.minimum(ypf / jnp.float32(14.0), jnp.float32(HF - 1))
        xfeat = jnp.minimum(xpf / jnp.float32(14.0), jnp.float32(WF - 1))
        ys = [rnd(jnp.clip(yfeat + jnp.float32(d - PATCH // 2),
                           jnp.float32(0.0), jnp.float32(HF - 1)))
              for d in range(PATCH)]
        xs = [rnd(jnp.clip(xfeat + jnp.float32(d - PATCH // 2),
                           jnp.float32(0.0), jnp.float32(WF - 1)))
              for d in range(PATCH)]
        slocal = g * L + lanes
        for p in range(P):
            row = tbase + ys[p // PATCH] * WF + xs[p % PATCH]
            plsc.store_scatter(idx_buf, [slocal, jnp.full((L,), p, jnp.int32)],
                               row)

    bufs = [buf0, buf1, buf2, buf3]

    def start_gather(c):
        return pltpu.async_copy(table_hbm.at[idx_buf.at[c]], bufs[c % NBUF],
                                gsem)

    gh = [None] * NCHUNK
    oh = [None] * NCHUNK
    for c in range(NBUF):
        gh[c] = start_gather(c)
    for c in range(NCHUNK):
        gh[c].wait()
        oh[c] = pltpu.async_copy(
            bufs[c % NBUF],
            out_hbm.at[pl.ds(out0 + c * ROWS_PER_CHUNK, ROWS_PER_CHUNK)],
            osem)
        if c + NBUF < NCHUNK:
            oh[c].wait()
            gh[c + NBUF] = start_gather(c + NBUF)
    for c in range(NCHUNK - NBUF, NCHUNK):
        oh[c].wait()


_sc_gather = functools.partial(
    pl.kernel,
    mesh=plsc.VectorSubcoreMesh(core_axis_name="c", subcore_axis_name="s"),
    out_type=jax.ShapeDtypeStruct((SAMP_TOTAL * P, C), jnp.float32),
    scratch_types=[
        pltpu.VMEM((SAMP_PER_W,), jnp.int32),
        pltpu.VMEM((SAMP_PER_W, P), jnp.int32),
        pltpu.VMEM((ROWS_PER_CHUNK, C), jnp.float32),
        pltpu.VMEM((ROWS_PER_CHUNK, C), jnp.float32),
        pltpu.VMEM((ROWS_PER_CHUNK, C), jnp.float32),
        pltpu.VMEM((ROWS_PER_CHUNK, C), jnp.float32),
        pltpu.SemaphoreType.DMA,
        pltpu.SemaphoreType.DMA,
    ],
    compiler_params=pltpu.CompilerParams(use_tc_tiling_on_sc=False,
                                         needs_layout_passes=False),
)(_sc_body)


def kernel(feature_maps, sampling_idx, heights, widths):
    del heights, widths
    table = jnp.transpose(feature_maps, (0, 2, 3, 1)).reshape(B * HF * WF, C)
    sflat = sampling_idx.reshape(SAMP_TOTAL)
    rows = _sc_gather(table, sflat)
    return rows.reshape(B, R, P * C)


# speedup vs baseline: 1.0036x; 1.0036x over previous
import functools

import jax
import jax.numpy as jnp
from jax import lax
from jax.experimental import pallas as pl
from jax.experimental.pallas import tpu as pltpu
from jax.experimental.pallas import tpu_sc as plsc

B = 4
C = 384
HF = 37
WF = 37
R = 512
P = 49
IMG_W = 518
PATCH = 7

NC = 2
NS = 16
NW = NC * NS
L = 16

SAMP_TOTAL = B * R
SAMP_PER_W = SAMP_TOTAL // NW
ROWS_PER_W = SAMP_PER_W * P
NBUF = 6
ROWS_PER_CHUNK = P
NCHUNK = SAMP_PER_W
W_PER_B = R // SAMP_PER_W

MAGIC = 12582912.0


def _sc_body(table_hbm, sidx_hbm, out_hbm, sidx_v, idx_buf, buf0, buf1,
             buf2, buf3, buf4, buf5, gsem, osem):
    wid = lax.axis_index("s") * NC + lax.axis_index("c")
    tbase = (wid // W_PER_B) * (HF * WF)
    s0 = wid * SAMP_PER_W
    out0 = wid * ROWS_PER_W

    pltpu.sync_copy(sidx_hbm.at[pl.ds(s0, SAMP_PER_W)], sidx_v)

    lanes = lax.iota(jnp.int32, L)

    def rnd(y):
        return ((y + jnp.float32(MAGIC)) - jnp.float32(MAGIC)).astype(
            jnp.int32)

    for g in range(SAMP_PER_W // L):
        s16 = sidx_v[pl.ds(g * L, L)]
        sf = s16.astype(jnp.float32)
        ypf = (sf / jnp.float32(IMG_W)).astype(jnp.int32).astype(jnp.float32)
        xpf = sf - ypf * jnp.float32(IMG_W)
        yfeat = jnp.minimum(ypf / jnp.float32(14.0), jnp.float32(HF - 1))
        xfeat = jnp.minimum(xpf / jnp.float32(14.0), jnp.float32(WF - 1))
        ys = [rnd(jnp.clip(yfeat + jnp.float32(d - PATCH // 2),
                           jnp.float32(0.0), jnp.float32(HF - 1)))
              for d in range(PATCH)]
        xs = [rnd(jnp.clip(xfeat + jnp.float32(d - PATCH // 2),
                           jnp.float32(0.0), jnp.float32(WF - 1)))
              for d in range(PATCH)]
        slocal = g * L + lanes
        for p in range(P):
            row = tbase + ys[p // PATCH] * WF + xs[p % PATCH]
            plsc.store_scatter(idx_buf, [slocal, jnp.full((L,), p, jnp.int32)],
                               row)

    bufs = [buf0, buf1, buf2, buf3, buf4, buf5]

    def start_gather(c):
        return pltpu.async_copy(table_hbm.at[idx_buf.at[c]], bufs[c % NBUF],
                                gsem)

    gh = [None] * NCHUNK
    oh = [None] * NCHUNK
    for c in range(NBUF):
        gh[c] = start_gather(c)
    for c in range(NCHUNK):
        gh[c].wait()
        oh[c] = pltpu.async_copy(
            bufs[c % NBUF],
            out_hbm.at[pl.ds(out0 + c * ROWS_PER_CHUNK, ROWS_PER_CHUNK)],
            osem)
        if c + NBUF < NCHUNK:
            oh[c].wait()
            gh[c + NBUF] = start_gather(c + NBUF)
    for c in range(NCHUNK - NBUF, NCHUNK):
        oh[c].wait()


_sc_gather = functools.partial(
    pl.kernel,
    mesh=plsc.VectorSubcoreMesh(core_axis_name="c", subcore_axis_name="s"),
    out_type=jax.ShapeDtypeStruct((SAMP_TOTAL * P, C), jnp.float32),
    scratch_types=[
        pltpu.VMEM((SAMP_PER_W,), jnp.int32),
        pltpu.VMEM((SAMP_PER_W, P), jnp.int32),
        pltpu.VMEM((ROWS_PER_CHUNK, C), jnp.float32),
        pltpu.VMEM((ROWS_PER_CHUNK, C), jnp.float32),
        pltpu.VMEM((ROWS_PER_CHUNK, C), jnp.float32),
        pltpu.VMEM((ROWS_PER_CHUNK, C), jnp.float32),
        pltpu.VMEM((ROWS_PER_CHUNK, C), jnp.float32),
        pltpu.VMEM((ROWS_PER_CHUNK, C), jnp.float32),
        pltpu.SemaphoreType.DMA,
        pltpu.SemaphoreType.DMA,
    ],
    compiler_params=pltpu.CompilerParams(use_tc_tiling_on_sc=False,
                                         needs_layout_passes=False),
)(_sc_body)


def kernel(feature_maps, sampling_idx, heights, widths):
    del heights, widths
    table = jnp.transpose(feature_maps, (0, 2, 3, 1)).reshape(B * HF * WF, C)
    sflat = sampling_idx.reshape(SAMP_TOTAL)
    rows = _sc_gather(table, sflat)
    return rows.reshape(B, R, P * C)
